# initial kernel scaffold (unmeasured)
import functools

import jax
import jax.numpy as jnp
from jax import lax
from jax.experimental import pallas as pl
from jax.experimental.pallas import tpu as pltpu

N_DEV = 8
SQ = 512
D = 1024
HQ = 8
DH = 128
SKV = 2048
CH = SQ // N_DEV
SCALE = 0.08838834764831843

N_STEPS = 2 * (N_DEV - 1)


def kernel(x, Wq, Wo, K_ext, V_ext):
    x2 = x.reshape(SQ, D)
    K = K_ext.reshape(SKV, HQ, DH)
    V = V_ext.reshape(SKV, HQ, DH)

    def body(x_ref, wq_ref, wo_ref, k_ref, v_ref, out_ref,
             comm_ref, send_sems, recv_sems):
        my = lax.axis_index("i")
        left = lax.rem(my - 1 + N_DEV, N_DEV)
        right = lax.rem(my + 1, N_DEV)

        barrier = pltpu.get_barrier_semaphore()
        for nbr in (left, right):
            pl.semaphore_signal(barrier, inc=1, device_id=(nbr,),
                                device_id_type=pl.DeviceIdType.MESH)
        pl.semaphore_wait(barrier, 2)

        xb = x_ref[:].astype(jnp.bfloat16)
        wqb = wq_ref[:].astype(jnp.bfloat16)
        q = jnp.dot(xb, wqb, preferred_element_type=jnp.float32)

        outs = []
        for h in range(HQ):
            qh = q[:, h * DH:(h + 1) * DH].astype(jnp.bfloat16)
            kh = k_ref[:, h, :].astype(jnp.bfloat16)
            vh = v_ref[:, h, :].astype(jnp.bfloat16)
            s = lax.dot_general(
                qh, kh, (((1,), (1,)), ((), ())),
                preferred_element_type=jnp.float32) * SCALE
            m = jnp.max(s, axis=-1, keepdims=True)
            p = jnp.exp(s - m)
            l = jnp.sum(p, axis=-1, keepdims=True)
            oh = jnp.dot(p.astype(jnp.bfloat16), vh,
                         preferred_element_type=jnp.float32) / l
            outs.append(oh)
        o = jnp.concatenate(outs, axis=1).astype(jnp.bfloat16)
        wob = wo_ref[:].astype(jnp.bfloat16)
        out_ref[:] = jnp.dot(o, wob, preferred_element_type=jnp.float32)

        for s in range(N_DEV - 1):
            send_idx = lax.rem(my - s + 8, 8)
            recv_idx = lax.rem(my - s - 1 + 8, 8)
            rdma = pltpu.make_async_remote_copy(
                src_ref=out_ref.at[pl.ds(send_idx * CH, CH), :],
                dst_ref=comm_ref.at[s],
                send_sem=send_sems.at[s],
                recv_sem=recv_sems.at[s],
                device_id=(right,),
                device_id_type=pl.DeviceIdType.MESH,
            )
            rdma.start()
            rdma.wait()
            out_ref[pl.ds(recv_idx * CH, CH), :] += comm_ref[s]

        for t in range(N_DEV - 1):
            slot = (N_DEV - 1) + t
            send_idx = lax.rem(my + 1 - t + 16, 8)
            recv_idx = lax.rem(my - t + 16, 8)
            rdma = pltpu.make_async_remote_copy(
                src_ref=out_ref.at[pl.ds(send_idx * CH, CH), :],
                dst_ref=comm_ref.at[slot],
                send_sem=send_sems.at[slot],
                recv_sem=recv_sems.at[slot],
                device_id=(right,),
                device_id_type=pl.DeviceIdType.MESH,
            )
            rdma.start()
            rdma.wait()
            out_ref[pl.ds(recv_idx * CH, CH), :] = comm_ref[slot]

        @functools.partial(pl.run_scoped,
                           second_barrier=pltpu.SemaphoreType.REGULAR)
        def _(second_barrier):
            for nbr in (left, right):
                pl.semaphore_signal(second_barrier, inc=1, device_id=(nbr,),
                                    device_id_type=pl.DeviceIdType.MESH)
            pl.semaphore_wait(second_barrier, 2)

    out = pl.pallas_call(
        body,
        out_shape=jax.ShapeDtypeStruct((SQ, D), jnp.float32),
        in_specs=[pl.BlockSpec(memory_space=pltpu.VMEM)] * 5,
        out_specs=pl.BlockSpec(memory_space=pltpu.VMEM),
        scratch_shapes=[
            pltpu.VMEM((N_STEPS, CH, D), jnp.float32),
            pltpu.SemaphoreType.DMA((N_STEPS,)),
            pltpu.SemaphoreType.DMA((N_STEPS,)),
        ],
        compiler_params=pltpu.CompilerParams(collective_id=0),
    )(x2, Wq, Wo, K, V)
    return out.reshape(1, SQ, D)


# baseline (device time: 122635 ns/iter reference)
import functools

import jax
import jax.numpy as jnp
from jax import lax
from jax.experimental import pallas as pl
from jax.experimental.pallas import tpu as pltpu

N_DEV = 8
SQ = 512
D = 1024
HQ = 8
DH = 128
SKV = 2048
CH = SQ // N_DEV
SCALE = 0.08838834764831843

N_STEPS = 2 * (N_DEV - 1)


def kernel(x, Wq, Wo, K_ext, V_ext):
    x2 = x.reshape(SQ, D).astype(jnp.bfloat16)
    Wq = Wq.astype(jnp.bfloat16)
    Wo = Wo.astype(jnp.bfloat16)
    K = K_ext.reshape(SKV, HQ, DH).astype(jnp.bfloat16)
    V = V_ext.reshape(SKV, HQ, DH).astype(jnp.bfloat16)

    def body(x_ref, wq_ref, wo_ref, k_ref, v_ref, out_ref,
             comm_ref, send_sems, recv_sems):
        my = lax.axis_index("i")
        left = lax.rem(my - 1 + N_DEV, N_DEV)
        right = lax.rem(my + 1, N_DEV)

        barrier = pltpu.get_barrier_semaphore()
        for nbr in (left, right):
            pl.semaphore_signal(barrier, inc=1, device_id=(nbr,),
                                device_id_type=pl.DeviceIdType.MESH)
        pl.semaphore_wait(barrier, 2)

        q = jnp.dot(x_ref[:], wq_ref[:],
                    preferred_element_type=jnp.float32)

        outs = []
        for h in range(HQ):
            qh = q[:, h * DH:(h + 1) * DH].astype(jnp.bfloat16)
            kh = k_ref[:, h, :]
            vh = v_ref[:, h, :]
            s = lax.dot_general(
                qh, kh, (((1,), (1,)), ((), ())),
                preferred_element_type=jnp.float32) * SCALE
            m = jnp.max(s, axis=-1, keepdims=True)
            p = jnp.exp(s - m)
            l = jnp.sum(p, axis=-1, keepdims=True)
            oh = jnp.dot(p.astype(jnp.bfloat16), vh,
                         preferred_element_type=jnp.float32) / l
            outs.append(oh)
        o = jnp.concatenate(outs, axis=1).astype(jnp.bfloat16)
        out_ref[:] = jnp.dot(o, wo_ref[:], preferred_element_type=jnp.float32)

        for s in range(N_DEV - 1):
            send_idx = lax.rem(my - s + 8, 8)
            recv_idx = lax.rem(my - s - 1 + 8, 8)
            rdma = pltpu.make_async_remote_copy(
                src_ref=out_ref.at[pl.ds(send_idx * CH, CH), :],
                dst_ref=comm_ref.at[s],
                send_sem=send_sems.at[s],
                recv_sem=recv_sems.at[s],
                device_id=(right,),
                device_id_type=pl.DeviceIdType.MESH,
            )
            rdma.start()
            rdma.wait()
            out_ref[pl.ds(recv_idx * CH, CH), :] += comm_ref[s]

        for t in range(N_DEV - 1):
            slot = (N_DEV - 1) + t
            send_idx = lax.rem(my + 1 - t + 16, 8)
            recv_idx = lax.rem(my - t + 16, 8)
            rdma = pltpu.make_async_remote_copy(
                src_ref=out_ref.at[pl.ds(send_idx * CH, CH), :],
                dst_ref=comm_ref.at[slot],
                send_sem=send_sems.at[slot],
                recv_sem=recv_sems.at[slot],
                device_id=(right,),
                device_id_type=pl.DeviceIdType.MESH,
            )
            rdma.start()
            rdma.wait()
            out_ref[pl.ds(recv_idx * CH, CH), :] = comm_ref[slot]

        @functools.partial(pl.run_scoped,
                           second_barrier=pltpu.SemaphoreType.REGULAR)
        def _(second_barrier):
            for nbr in (left, right):
                pl.semaphore_signal(second_barrier, inc=1, device_id=(nbr,),
                                    device_id_type=pl.DeviceIdType.MESH)
            pl.semaphore_wait(second_barrier, 2)

    out = pl.pallas_call(
        body,
        out_shape=jax.ShapeDtypeStruct((SQ, D), jnp.float32),
        in_specs=[pl.BlockSpec(memory_space=pltpu.VMEM)] * 5,
        out_specs=pl.BlockSpec(memory_space=pltpu.VMEM),
        scratch_shapes=[
            pltpu.VMEM((N_STEPS, CH, D), jnp.float32),
            pltpu.SemaphoreType.DMA((N_STEPS,)),
            pltpu.SemaphoreType.DMA((N_STEPS,)),
        ],
        compiler_params=pltpu.CompilerParams(
            collective_id=0, vmem_limit_bytes=64 * 1024 * 1024),
    )(x2, Wq, Wo, K, V)
    return out.reshape(1, SQ, D)
